# baseline (speedup 1.0000x reference)
import functools

import jax
import jax.numpy as jnp
from jax.experimental import pallas as pl
from jax.experimental.pallas import tpu as pltpu

B, S, H = 2, 2048, 1024
E = 8
TOPK = 2
I = 2048
T = B * S

BTR = 512
BT = 256
BI = 512
NI = I // BI


def _router_body(x_ref, gw_ref, idx_ref, w_ref):
    xb = x_ref[...]
    gw = gw_ref[...]
    logits = jax.lax.dot_general(xb, gw, (((1,), (1,)), ((), ())),
                                 preferred_element_type=jnp.float32)
    m = jnp.max(logits, axis=-1, keepdims=True)
    p = jnp.exp(logits - m)
    p = p / jnp.sum(p, axis=-1, keepdims=True)
    iota = jax.lax.broadcasted_iota(jnp.int32, p.shape, 1)
    m0 = jnp.max(p, axis=-1, keepdims=True)
    i0 = jnp.min(jnp.where(p == m0, iota, E), axis=-1, keepdims=True)
    p2 = jnp.where(iota == i0, -jnp.inf, p)
    m1 = jnp.max(p2, axis=-1, keepdims=True)
    i1 = jnp.min(jnp.where(p2 == m1, iota, E), axis=-1, keepdims=True)
    s = m0 + m1 + 1e-20
    idx_ref[...] = jnp.concatenate([i0, i1], axis=1)
    w_ref[...] = jnp.concatenate([m0 / s, m1 / s], axis=1)


def _router(x_flat, gate_W):
    return pl.pallas_call(
        _router_body,
        grid=(T // BTR,),
        in_specs=[
            pl.BlockSpec((BTR, H), lambda b: (b, 0)),
            pl.BlockSpec((E, H), lambda b: (0, 0)),
        ],
        out_specs=[
            pl.BlockSpec((BTR, TOPK), lambda b: (b, 0)),
            pl.BlockSpec((BTR, TOPK), lambda b: (b, 0)),
        ],
        out_shape=[
            jax.ShapeDtypeStruct((T, TOPK), jnp.int32),
            jax.ShapeDtypeStruct((T, TOPK), jnp.float32),
        ],
    )(x_flat, gate_W)


def _silu(a):
    return a * jax.nn.sigmoid(a)


def _ff_partial(xb, wg_t, wu_t, wd_t):
    a = jax.lax.dot_general(xb, wg_t, (((1,), (1,)), ((), ())),
                            preferred_element_type=jnp.float32)
    u = jax.lax.dot_general(xb, wu_t, (((1,), (1,)), ((), ())),
                            preferred_element_type=jnp.float32)
    h = _silu(a) * u
    return jax.lax.dot_general(h, wd_t, (((1,), (1,)), ((), ())),
                               preferred_element_type=jnp.float32)


def _shared_body(x_ref, wg_ref, wu_ref, wd_ref, out_ref):
    i = pl.program_id(1)
    part = _ff_partial(x_ref[...], wg_ref[...], wu_ref[...], wd_ref[...])

    @pl.when(i == 0)
    def _():
        out_ref[...] = part

    @pl.when(i != 0)
    def _():
        out_ref[...] += part


def _shared_ff(x_flat, Wg_s, Wu_s, Wd_s):
    return pl.pallas_call(
        _shared_body,
        grid=(T // BT, NI),
        in_specs=[
            pl.BlockSpec((BT, H), lambda b, i: (b, 0)),
            pl.BlockSpec((BI, H), lambda b, i: (i, 0)),
            pl.BlockSpec((BI, H), lambda b, i: (i, 0)),
            pl.BlockSpec((H, BI), lambda b, i: (0, i)),
        ],
        out_specs=pl.BlockSpec((BT, H), lambda b, i: (b, 0)),
        out_shape=jax.ShapeDtypeStruct((T, H), jnp.float32),
    )(x_flat, Wg_s, Wu_s, Wd_s)


def _dense_routed_body(x_ref, idx_ref, w_ref, ys_ref, wg_ref, wu_ref, wd_ref,
                       out_ref):
    e = pl.program_id(1)
    i = pl.program_id(2)
    xb = x_ref[...]
    part = _ff_partial(xb, wg_ref[0], wu_ref[0], wd_ref[0])
    idx = idx_ref[...]
    w = w_ref[...]
    we = jnp.sum(jnp.where(idx == e, w, 0.0), axis=1, keepdims=True)
    part = part * we

    @pl.when((e == 0) & (i == 0))
    def _():
        out_ref[...] = ys_ref[...] + part

    @pl.when((e != 0) | (i != 0))
    def _():
        out_ref[...] += part


def _dense_routed(x_flat, idx, w, y_s, Wg, Wu, Wd):
    return pl.pallas_call(
        _dense_routed_body,
        grid=(T // BT, E, NI),
        in_specs=[
            pl.BlockSpec((BT, H), lambda b, e, i: (b, 0)),
            pl.BlockSpec((BT, TOPK), lambda b, e, i: (b, 0)),
            pl.BlockSpec((BT, TOPK), lambda b, e, i: (b, 0)),
            pl.BlockSpec((BT, H), lambda b, e, i: (b, 0)),
            pl.BlockSpec((1, BI, H), lambda b, e, i: (e, i, 0)),
            pl.BlockSpec((1, BI, H), lambda b, e, i: (e, i, 0)),
            pl.BlockSpec((1, H, BI), lambda b, e, i: (e, 0, i)),
        ],
        out_specs=pl.BlockSpec((BT, H), lambda b, e, i: (b, 0)),
        out_shape=jax.ShapeDtypeStruct((T, H), jnp.float32),
    )(x_flat, idx, w, y_s, Wg, Wu, Wd)


def kernel(x, Wg_s, Wu_s, Wd_s, gate_W, Wg, Wu, Wd):
    x_flat = x.reshape(T, H)
    idx, w = _router(x_flat, gate_W)
    y_s = _shared_ff(x_flat, Wg_s, Wu_s, Wd_s)
    out = _dense_routed(x_flat, idx, w, y_s, Wg, Wu, Wd)
    return out.reshape(B, S, H)


# speedup vs baseline: 2.7680x; 2.7680x over previous
import functools

import jax
import jax.numpy as jnp
from jax import lax
from jax.experimental import pallas as pl
from jax.experimental.pallas import tpu as pltpu
from jax.experimental.pallas import tpu_sc as plsc

B, S, H = 2, 2048, 1024
E = 8
TOPK = 2
I = 2048
T = B * S
NP = T * TOPK

BTR = 512
BT = 256
BI = 512
NI = I // BI

NBLK = NP // BT + E
NR = NBLK * BT
META = 64

NC, NS, NLANE = 2, 16, 16
NW = NC * NS

_mesh = functools.partial(
    plsc.VectorSubcoreMesh, core_axis_name="c", subcore_axis_name="s")


def _wid():
    return lax.axis_index("s") * NC + lax.axis_index("c")


def _router_body(x_ref, gw_ref, idx_ref, w_ref):
    xb = x_ref[...]
    gw = gw_ref[...]
    logits = lax.dot_general(xb, gw, (((1,), (1,)), ((), ())),
                             preferred_element_type=jnp.float32)
    m = jnp.max(logits, axis=-1, keepdims=True)
    p = jnp.exp(logits - m)
    p = p / jnp.sum(p, axis=-1, keepdims=True)
    iota = lax.broadcasted_iota(jnp.int32, p.shape, 1)
    m0 = jnp.max(p, axis=-1, keepdims=True)
    i0 = jnp.min(jnp.where(p == m0, iota, E), axis=-1, keepdims=True)
    p2 = jnp.where(iota == i0, -jnp.inf, p)
    m1 = jnp.max(p2, axis=-1, keepdims=True)
    i1 = jnp.min(jnp.where(p2 == m1, iota, E), axis=-1, keepdims=True)
    s = m0 + m1 + 1e-20
    idx_ref[...] = jnp.concatenate([i0, i1], axis=1)
    w_ref[...] = jnp.concatenate([m0 / s, m1 / s], axis=1)


def _router(x_flat, gate_W):
    return pl.pallas_call(
        _router_body,
        grid=(T // BTR,),
        in_specs=[
            pl.BlockSpec((BTR, H), lambda b: (b, 0)),
            pl.BlockSpec((E, H), lambda b: (0, 0)),
        ],
        out_specs=[
            pl.BlockSpec((BTR, TOPK), lambda b: (b, 0)),
            pl.BlockSpec((BTR, TOPK), lambda b: (b, 0)),
        ],
        out_shape=[
            jax.ShapeDtypeStruct((T, TOPK), jnp.int32),
            jax.ShapeDtypeStruct((T, TOPK), jnp.float32),
        ],
    )(x_flat, gate_W)


def _plan_body(tki_hbm, tkw_hbm, meta_hbm, stok_hbm, pos_hbm, sw_hbm,
               idx_v, w_v, pos_v, stok_v, sw_v, meta_v, tmp_v):
    w = _wid()

    @pl.when(w == 0)
    def _():
        pltpu.sync_copy(tki_hbm, idx_v)
        pltpu.sync_copy(tkw_hbm, w_v)

        zi = jnp.zeros((NLANE,), jnp.int32)
        zf = jnp.zeros((NLANE,), jnp.float32)

        def fill(j, _):
            stok_v[pl.ds(j * NLANE, NLANE)] = zi
            sw_v[pl.ds(j * NLANE, NLANE)] = zf
            return 0
        lax.fori_loop(0, NR // NLANE, fill, 0)

        def hist(v, acc):
            x16 = idx_v[pl.ds(v * NLANE, NLANE)]
            return tuple(acc[e] + jnp.where(x16 == e, 1, 0).astype(jnp.int32)
                         for e in range(E))
        acc0 = tuple(jnp.zeros((NLANE,), jnp.int32) for _ in range(E))
        acc = lax.fori_loop(0, NP // NLANE, hist, acc0)
        counts = [plsc.cumsum(acc[e])[NLANE - 1] for e in range(E)]

        lane = lax.iota(jnp.int32, NLANE)
        run_slots = jnp.int32(0)
        nblk = jnp.int32(0)
        last_e = jnp.int32(0)
        blk_end = []
        cur0 = []
        for e in range(E):
            c = counts[e]
            nb_e = (c + (BT - 1)) >> 8
            cur0.append(jnp.full((NLANE,), run_slots, jnp.int32))
            run_slots = run_slots + (nb_e << 8)
            nblk = nblk + nb_e
            blk_end.append(nblk)
            last_e = jnp.where(nb_e > 0, jnp.int32(e), last_e)

        for v in range(META // NLANE):
            bvec = lane + v * NLANE
            cnt = jnp.zeros((NLANE,), jnp.int32)
            for e in range(E):
                cnt = cnt + jnp.where(bvec >= blk_end[e], 1, 0).astype(jnp.int32)
            mv = jnp.minimum(cnt, last_e)
            if v == META // NLANE - 1:
                mv = jnp.where(lane == NLANE - 1, nblk, mv)
            meta_v[pl.ds(v * NLANE, NLANE)] = mv

        def assign(v, cur):
            x16 = idx_v[pl.ds(v * NLANE, NLANE)]
            w16 = w_v[pl.ds(v * NLANE, NLANE)]
            tok16 = (lane + v * NLANE) >> 1
            slot16 = jnp.zeros((NLANE,), jnp.int32)
            ncur = []
            for e in range(E):
                m = x16 == e
                mi = m.astype(jnp.int32)
                rank = plsc.cumsum(mi) - mi
                slot16 = jnp.where(m, cur[e] + rank, slot16)
                ncur.append(cur[e] + plsc.all_reduce_population_count(m))
            pos_v[pl.ds(v * NLANE, NLANE)] = slot16
            plsc.store_scatter(stok_v, [slot16], tok16)
            plsc.store_scatter(sw_v, [slot16], w16)
            return tuple(ncur)
        lax.fori_loop(0, NP // NLANE, assign, tuple(cur0))

        pltpu.sync_copy(meta_v, meta_hbm)
        pltpu.sync_copy(stok_v, stok_hbm)
        pltpu.sync_copy(pos_v, pos_hbm)
        pltpu.sync_copy(sw_v, sw_hbm)


def _plan(tk_idx, tk_w):
    return pl.kernel(
        _plan_body,
        out_type=[
            jax.ShapeDtypeStruct((META,), jnp.int32),
            jax.ShapeDtypeStruct((NR,), jnp.int32),
            jax.ShapeDtypeStruct((NP,), jnp.int32),
            jax.ShapeDtypeStruct((NR,), jnp.float32),
        ],
        mesh=_mesh(),
        scratch_types=[
            pltpu.VMEM((NP,), jnp.int32),
            pltpu.VMEM((NP,), jnp.float32),
            pltpu.VMEM((NP,), jnp.int32),
            pltpu.VMEM((NR,), jnp.int32),
            pltpu.VMEM((NR,), jnp.float32),
            pltpu.VMEM((META,), jnp.int32),
            pltpu.VMEM((NLANE,), jnp.int32),
        ],
        compiler_params=pltpu.CompilerParams(needs_layout_passes=False),
    )(tk_idx, tk_w)


_GROWS = NR // NW
_GCH = _GROWS // NLANE


def _gather_body(x_hbm, stok_hbm, gx_hbm, sidx_v, buf0, buf1, sem0, sem1):
    w = _wid()
    base = w * _GROWS
    pltpu.sync_copy(stok_hbm.at[pl.ds(base, _GROWS)], sidx_v)
    bufs = (buf0, buf1)
    sems = (sem0, sem1)
    copies = [None, None]
    for c in range(_GCH):
        iv = sidx_v[pl.ds(c * NLANE, NLANE)]
        copies[c % 2] = pltpu.async_copy(x_hbm.at[iv], bufs[c % 2], sems[c % 2])
        if c > 0:
            copies[(c - 1) % 2].wait()
            pltpu.sync_copy(bufs[(c - 1) % 2],
                            gx_hbm.at[pl.ds(base + (c - 1) * NLANE, NLANE)])
    copies[(_GCH - 1) % 2].wait()
    pltpu.sync_copy(bufs[(_GCH - 1) % 2],
                    gx_hbm.at[pl.ds(base + (_GCH - 1) * NLANE, NLANE)])


def _gather_x(x_flat, stok):
    return pl.kernel(
        _gather_body,
        out_type=jax.ShapeDtypeStruct((NR, H), jnp.float32),
        mesh=_mesh(),
        scratch_types=[
            pltpu.VMEM((_GROWS,), jnp.int32),
            pltpu.VMEM((NLANE, H), jnp.float32),
            pltpu.VMEM((NLANE, H), jnp.float32),
            pltpu.SemaphoreType.DMA,
            pltpu.SemaphoreType.DMA,
        ],
        compiler_params=pltpu.CompilerParams(needs_layout_passes=False),
    )(x_flat, stok)


def _silu(a):
    return a * jax.nn.sigmoid(a)


def _ff_chunk(xb, wg_t, wu_t, wd_t):
    a = lax.dot_general(xb, wg_t, (((1,), (1,)), ((), ())),
                        preferred_element_type=jnp.float32)
    u = lax.dot_general(xb, wu_t, (((1,), (1,)), ((), ())),
                        preferred_element_type=jnp.float32)
    h = _silu(a) * u
    return lax.dot_general(h, wd_t, (((1,), (1,)), ((), ())),
                           preferred_element_type=jnp.float32)


def _shared_body(x_ref, wg_ref, wu_ref, wd_ref, out_ref):
    xb = x_ref[...]
    acc = _ff_chunk(xb, wg_ref[pl.ds(0, BI), :], wu_ref[pl.ds(0, BI), :],
                    wd_ref[:, pl.ds(0, BI)])
    for ic in range(1, NI):
        acc += _ff_chunk(xb, wg_ref[pl.ds(ic * BI, BI), :],
                         wu_ref[pl.ds(ic * BI, BI), :],
                         wd_ref[:, pl.ds(ic * BI, BI)])
    out_ref[...] = acc


def _shared_ff(x_flat, Wg_s, Wu_s, Wd_s):
    return pl.pallas_call(
        _shared_body,
        grid=(T // BT,),
        in_specs=[
            pl.BlockSpec((BT, H), lambda b: (b, 0)),
            pl.BlockSpec((I, H), lambda b: (0, 0)),
            pl.BlockSpec((I, H), lambda b: (0, 0)),
            pl.BlockSpec((H, I), lambda b: (0, 0)),
        ],
        out_specs=pl.BlockSpec((BT, H), lambda b: (b, 0)),
        out_shape=jax.ShapeDtypeStruct((T, H), jnp.float32),
    )(x_flat, Wg_s, Wu_s, Wd_s)


def _grouped_body(meta_ref, gx_ref, sw_ref, wg_ref, wu_ref, wd_ref, y_ref):
    b = pl.program_id(0)
    nblk = meta_ref[META - 1]

    @pl.when(b < nblk)
    def _():
        xb = gx_ref[...]
        acc = _ff_chunk(xb, wg_ref[0, pl.ds(0, BI), :],
                        wu_ref[0, pl.ds(0, BI), :], wd_ref[0, :, pl.ds(0, BI)])
        for ic in range(1, NI):
            acc += _ff_chunk(xb, wg_ref[0, pl.ds(ic * BI, BI), :],
                             wu_ref[0, pl.ds(ic * BI, BI), :],
                             wd_ref[0, :, pl.ds(ic * BI, BI)])
        y_ref[...] = acc * sw_ref[0, 0, :][:, None]


def _grouped_ff(meta, gx, sw2d, Wg, Wu, Wd):
    grid_spec = pltpu.PrefetchScalarGridSpec(
        num_scalar_prefetch=1,
        grid=(NBLK,),
        in_specs=[
            pl.BlockSpec((BT, H), lambda b, m: (b, 0)),
            pl.BlockSpec((1, 1, BT), lambda b, m: (b, 0, 0)),
            pl.BlockSpec((1, I, H), lambda b, m: (m[b], 0, 0)),
            pl.BlockSpec((1, I, H), lambda b, m: (m[b], 0, 0)),
            pl.BlockSpec((1, H, I), lambda b, m: (m[b], 0, 0)),
        ],
        out_specs=pl.BlockSpec((BT, H), lambda b, m: (b, 0)),
    )
    return pl.pallas_call(
        _grouped_body,
        grid_spec=grid_spec,
        out_shape=jax.ShapeDtypeStruct((NR, H), jnp.float32),
    )(meta, gx, sw2d, Wg, Wu, Wd)


_CROWS = T // NW
_CCH = _CROWS // NLANE
_VH = H // NLANE


def _combine_body(ys_hbm, yr_hbm, pos_hbm, out_hbm,
                  pos_v, bufs, bufr0, bufr1, bufo, sems, semr0, semr1):
    w = _wid()
    tbase = w * _CROWS
    pltpu.sync_copy(pos_hbm.at[pl.ds(tbase * 2, _CROWS * 2)], pos_v)
    for c in range(_CCH):
        pa = pos_v[pl.ds(c * 2 * NLANE, NLANE)]
        pb = pos_v[pl.ds(c * 2 * NLANE + NLANE, NLANE)]
        cs = pltpu.async_copy(ys_hbm.at[pl.ds(tbase + c * NLANE, NLANE)],
                              bufs, sems)
        c0 = pltpu.async_copy(yr_hbm.at[pa], bufr0, semr0)
        c1 = pltpu.async_copy(yr_hbm.at[pb], bufr1, semr1)
        cs.wait()
        c0.wait()
        c1.wait()

        def vsum(j, _):
            for t in range(NLANE):
                br = bufr0 if t < NLANE // 2 else bufr1
                u = 2 * t if t < NLANE // 2 else 2 * (t - NLANE // 2)
                bufo[t, pl.ds(j * NLANE, NLANE)] = (
                    bufs[t, pl.ds(j * NLANE, NLANE)]
                    + br[u, pl.ds(j * NLANE, NLANE)]
                    + br[u + 1, pl.ds(j * NLANE, NLANE)])
            return 0
        lax.fori_loop(0, _VH, vsum, 0)
        pltpu.sync_copy(bufo, out_hbm.at[pl.ds(tbase + c * NLANE, NLANE)])


def _combine(y_s, y_r, pos):
    return pl.kernel(
        _combine_body,
        out_type=jax.ShapeDtypeStruct((T, H), jnp.float32),
        mesh=_mesh(),
        scratch_types=[
            pltpu.VMEM((_CROWS * 2,), jnp.int32),
            pltpu.VMEM((NLANE, H), jnp.float32),
            pltpu.VMEM((NLANE, H), jnp.float32),
            pltpu.VMEM((NLANE, H), jnp.float32),
            pltpu.VMEM((NLANE, H), jnp.float32),
            pltpu.SemaphoreType.DMA,
            pltpu.SemaphoreType.DMA,
            pltpu.SemaphoreType.DMA,
        ],
        compiler_params=pltpu.CompilerParams(needs_layout_passes=False),
    )(y_s, y_r, pos)


def kernel(x, Wg_s, Wu_s, Wd_s, gate_W, Wg, Wu, Wd):
    x_flat = x.reshape(T, H)
    idx, w = _router(x_flat, gate_W)
    meta, stok, pos, sw = _plan(idx.reshape(-1), w.reshape(-1))
    gx = _gather_x(x_flat, stok)
    y_s = _shared_ff(x_flat, Wg_s, Wu_s, Wd_s)
    y_r = _grouped_ff(meta, gx, sw.reshape(NBLK, 1, BT), Wg, Wu, Wd)
    out = _combine(y_s, y_r, pos)
    return out.reshape(B, S, H)


# speedup vs baseline: 2.8823x; 1.0413x over previous
import functools

import jax
import jax.numpy as jnp
from jax import lax
from jax.experimental import pallas as pl
from jax.experimental.pallas import tpu as pltpu
from jax.experimental.pallas import tpu_sc as plsc

B, S, H = 2, 2048, 1024
E = 8
TOPK = 2
I = 2048
T = B * S
NP = T * TOPK

BTR = 512
BT = 256
BI = 512
NI = I // BI

NBLK = NP // BT + E
NR = NBLK * BT
META = 64

NC, NS, NLANE = 2, 16, 16
NW = NC * NS

_mesh = functools.partial(
    plsc.VectorSubcoreMesh, core_axis_name="c", subcore_axis_name="s")


def _wid():
    return lax.axis_index("s") * NC + lax.axis_index("c")


def _router_body(x_ref, gw_ref, idx_ref, w_ref):
    xb = x_ref[...]
    gw = gw_ref[...]
    logits = lax.dot_general(xb, gw, (((1,), (1,)), ((), ())),
                             preferred_element_type=jnp.float32)
    m = jnp.max(logits, axis=-1, keepdims=True)
    p = jnp.exp(logits - m)
    p = p / jnp.sum(p, axis=-1, keepdims=True)
    iota = lax.broadcasted_iota(jnp.int32, p.shape, 1)
    m0 = jnp.max(p, axis=-1, keepdims=True)
    i0 = jnp.min(jnp.where(p == m0, iota, E), axis=-1, keepdims=True)
    p2 = jnp.where(iota == i0, -jnp.inf, p)
    m1 = jnp.max(p2, axis=-1, keepdims=True)
    i1 = jnp.min(jnp.where(p2 == m1, iota, E), axis=-1, keepdims=True)
    s = m0 + m1 + 1e-20
    idx_ref[...] = jnp.concatenate([i0, i1], axis=1)
    w_ref[...] = jnp.concatenate([m0 / s, m1 / s], axis=1)


def _router(x_flat, gate_W):
    return pl.pallas_call(
        _router_body,
        grid=(T // BTR,),
        in_specs=[
            pl.BlockSpec((BTR, H), lambda b: (b, 0)),
            pl.BlockSpec((E, H), lambda b: (0, 0)),
        ],
        out_specs=[
            pl.BlockSpec((BTR, TOPK), lambda b: (b, 0)),
            pl.BlockSpec((BTR, TOPK), lambda b: (b, 0)),
        ],
        out_shape=[
            jax.ShapeDtypeStruct((T, TOPK), jnp.int32),
            jax.ShapeDtypeStruct((T, TOPK), jnp.float32),
        ],
    )(x_flat, gate_W)


def _plan_body(tki_hbm, tkw_hbm, meta_hbm, stok_hbm, pos_hbm, sw_hbm,
               idx_v, w_v, pos_v, stok_v, sw_v, meta_v, tmp_v):
    w = _wid()

    @pl.when(w == 0)
    def _():
        pltpu.sync_copy(tki_hbm, idx_v)
        pltpu.sync_copy(tkw_hbm, w_v)

        zi = jnp.zeros((NLANE,), jnp.int32)
        zf = jnp.zeros((NLANE,), jnp.float32)

        def fill(j, _):
            stok_v[pl.ds(j * NLANE, NLANE)] = zi
            sw_v[pl.ds(j * NLANE, NLANE)] = zf
            return 0
        lax.fori_loop(0, NR // NLANE, fill, 0)

        def hist(v, acc):
            x16 = idx_v[pl.ds(v * NLANE, NLANE)]
            return tuple(acc[e] + jnp.where(x16 == e, 1, 0).astype(jnp.int32)
                         for e in range(E))
        acc0 = tuple(jnp.zeros((NLANE,), jnp.int32) for _ in range(E))
        acc = lax.fori_loop(0, NP // NLANE, hist, acc0)
        counts = [plsc.cumsum(acc[e])[NLANE - 1] for e in range(E)]

        lane = lax.iota(jnp.int32, NLANE)
        run_slots = jnp.int32(0)
        nblk = jnp.int32(0)
        last_e = jnp.int32(0)
        blk_end = []
        cur0 = []
        for e in range(E):
            c = counts[e]
            nb_e = (c + (BT - 1)) >> 8
            cur0.append(jnp.full((NLANE,), run_slots, jnp.int32))
            run_slots = run_slots + (nb_e << 8)
            nblk = nblk + nb_e
            blk_end.append(nblk)
            last_e = jnp.where(nb_e > 0, jnp.int32(e), last_e)

        for v in range(META // NLANE):
            bvec = lane + v * NLANE
            cnt = jnp.zeros((NLANE,), jnp.int32)
            for e in range(E):
                cnt = cnt + jnp.where(bvec >= blk_end[e], 1, 0).astype(jnp.int32)
            mv = jnp.minimum(cnt, last_e)
            if v == META // NLANE - 1:
                mv = jnp.where(lane == NLANE - 1, nblk, mv)
            meta_v[pl.ds(v * NLANE, NLANE)] = mv

        def assign(v, cur):
            x16 = idx_v[pl.ds(v * NLANE, NLANE)]
            w16 = w_v[pl.ds(v * NLANE, NLANE)]
            tok16 = (lane + v * NLANE) >> 1
            slot16 = jnp.zeros((NLANE,), jnp.int32)
            ncur = []
            for e in range(E):
                m = x16 == e
                mi = m.astype(jnp.int32)
                rank = plsc.cumsum(mi) - mi
                slot16 = jnp.where(m, cur[e] + rank, slot16)
                ncur.append(cur[e] + plsc.all_reduce_population_count(m))
            pos_v[pl.ds(v * NLANE, NLANE)] = slot16
            plsc.store_scatter(stok_v, [slot16], tok16)
            plsc.store_scatter(sw_v, [slot16], w16)
            return tuple(ncur)
        lax.fori_loop(0, NP // NLANE, assign, tuple(cur0))

        pltpu.sync_copy(meta_v, meta_hbm)
        pltpu.sync_copy(stok_v, stok_hbm)
        pltpu.sync_copy(pos_v, pos_hbm)
        pltpu.sync_copy(sw_v, sw_hbm)


def _plan(tk_idx, tk_w):
    return pl.kernel(
        _plan_body,
        out_type=[
            jax.ShapeDtypeStruct((META,), jnp.int32),
            jax.ShapeDtypeStruct((NR,), jnp.int32),
            jax.ShapeDtypeStruct((NP,), jnp.int32),
            jax.ShapeDtypeStruct((NR,), jnp.float32),
        ],
        mesh=_mesh(),
        scratch_types=[
            pltpu.VMEM((NP,), jnp.int32),
            pltpu.VMEM((NP,), jnp.float32),
            pltpu.VMEM((NP,), jnp.int32),
            pltpu.VMEM((NR,), jnp.int32),
            pltpu.VMEM((NR,), jnp.float32),
            pltpu.VMEM((META,), jnp.int32),
            pltpu.VMEM((NLANE,), jnp.int32),
        ],
        compiler_params=pltpu.CompilerParams(needs_layout_passes=False),
    )(tk_idx, tk_w)


_GROWS = NR // NW
_GCH = _GROWS // NLANE


def _gather_body(x_hbm, stok_hbm, gx_hbm, sidx_v, buf0, buf1,
                 gsem0, gsem1, wsem0, wsem1):
    w = _wid()
    base = w * _GROWS
    pltpu.sync_copy(stok_hbm.at[pl.ds(base, _GROWS)], sidx_v)
    bufs = (buf0, buf1)
    gsems = (gsem0, gsem1)
    wsems = (wsem0, wsem1)
    gcop = [None, None]
    wcop = [None, None]
    iv0 = sidx_v[pl.ds(0, NLANE)]
    gcop[0] = pltpu.async_copy(x_hbm.at[iv0], bufs[0], gsems[0])
    for c in range(_GCH):
        b = c % 2
        gcop[b].wait()
        wcop[b] = pltpu.async_copy(
            bufs[b], gx_hbm.at[pl.ds(base + c * NLANE, NLANE)], wsems[b])
        if c + 1 < _GCH:
            nb = (c + 1) % 2
            if wcop[nb] is not None:
                wcop[nb].wait()
            iv = sidx_v[pl.ds((c + 1) * NLANE, NLANE)]
            gcop[nb] = pltpu.async_copy(x_hbm.at[iv], bufs[nb], gsems[nb])
    wcop[0].wait()
    wcop[1].wait()


def _gather_x(x_flat, stok):
    return pl.kernel(
        _gather_body,
        out_type=jax.ShapeDtypeStruct((NR, H), jnp.float32),
        mesh=_mesh(),
        scratch_types=[
            pltpu.VMEM((_GROWS,), jnp.int32),
            pltpu.VMEM((NLANE, H), jnp.float32),
            pltpu.VMEM((NLANE, H), jnp.float32),
            pltpu.SemaphoreType.DMA,
            pltpu.SemaphoreType.DMA,
            pltpu.SemaphoreType.DMA,
            pltpu.SemaphoreType.DMA,
        ],
        compiler_params=pltpu.CompilerParams(needs_layout_passes=False),
    )(x_flat, stok)


def _silu(a):
    return a * jax.nn.sigmoid(a)


def _ff_chunk(xb, wg_t, wu_t, wd_t):
    a = lax.dot_general(xb, wg_t, (((1,), (1,)), ((), ())),
                        preferred_element_type=jnp.float32)
    u = lax.dot_general(xb, wu_t, (((1,), (1,)), ((), ())),
                        preferred_element_type=jnp.float32)
    h = _silu(a) * u
    return lax.dot_general(h, wd_t, (((1,), (1,)), ((), ())),
                           preferred_element_type=jnp.float32)


def _shared_body(x_ref, wg_ref, wu_ref, wd_ref, out_ref):
    xb = x_ref[...]
    acc = _ff_chunk(xb, wg_ref[pl.ds(0, BI), :], wu_ref[pl.ds(0, BI), :],
                    wd_ref[:, pl.ds(0, BI)])
    for ic in range(1, NI):
        acc += _ff_chunk(xb, wg_ref[pl.ds(ic * BI, BI), :],
                         wu_ref[pl.ds(ic * BI, BI), :],
                         wd_ref[:, pl.ds(ic * BI, BI)])
    out_ref[...] = acc


def _shared_ff(x_flat, Wg_s, Wu_s, Wd_s):
    return pl.pallas_call(
        _shared_body,
        grid=(T // BT,),
        in_specs=[
            pl.BlockSpec((BT, H), lambda b: (b, 0)),
            pl.BlockSpec((I, H), lambda b: (0, 0)),
            pl.BlockSpec((I, H), lambda b: (0, 0)),
            pl.BlockSpec((H, I), lambda b: (0, 0)),
        ],
        out_specs=pl.BlockSpec((BT, H), lambda b: (b, 0)),
        out_shape=jax.ShapeDtypeStruct((T, H), jnp.float32),
    )(x_flat, Wg_s, Wu_s, Wd_s)


def _grouped_body(meta_ref, gx_ref, sw_ref, wg_ref, wu_ref, wd_ref, y_ref):
    b = pl.program_id(0)
    nblk = meta_ref[META - 1]

    @pl.when(b < nblk)
    def _():
        xb = gx_ref[...]
        acc = _ff_chunk(xb, wg_ref[0, pl.ds(0, BI), :],
                        wu_ref[0, pl.ds(0, BI), :], wd_ref[0, :, pl.ds(0, BI)])
        for ic in range(1, NI):
            acc += _ff_chunk(xb, wg_ref[0, pl.ds(ic * BI, BI), :],
                             wu_ref[0, pl.ds(ic * BI, BI), :],
                             wd_ref[0, :, pl.ds(ic * BI, BI)])
        y_ref[...] = acc * sw_ref[0, 0, :][:, None]


def _grouped_ff(meta, gx, sw2d, Wg, Wu, Wd):
    grid_spec = pltpu.PrefetchScalarGridSpec(
        num_scalar_prefetch=1,
        grid=(NBLK,),
        in_specs=[
            pl.BlockSpec((BT, H), lambda b, m: (b, 0)),
            pl.BlockSpec((1, 1, BT), lambda b, m: (b, 0, 0)),
            pl.BlockSpec((1, I, H), lambda b, m: (m[b], 0, 0)),
            pl.BlockSpec((1, I, H), lambda b, m: (m[b], 0, 0)),
            pl.BlockSpec((1, H, I), lambda b, m: (m[b], 0, 0)),
        ],
        out_specs=pl.BlockSpec((BT, H), lambda b, m: (b, 0)),
    )
    return pl.pallas_call(
        _grouped_body,
        grid_spec=grid_spec,
        out_shape=jax.ShapeDtypeStruct((NR, H), jnp.float32),
    )(meta, gx, sw2d, Wg, Wu, Wd)


_CROWS = T // NW
_CCH = _CROWS // NLANE
_VH = H // NLANE


def _combine_body(ys_hbm, yr_hbm, pos_hbm, out_hbm,
                  pos_v, bufs0, bufs1, bufr00, bufr01, bufr10, bufr11, bufo,
                  sems0, sems1, semr00, semr01, semr10, semr11):
    w = _wid()
    tbase = w * _CROWS
    pltpu.sync_copy(pos_hbm.at[pl.ds(tbase * 2, _CROWS * 2)], pos_v)
    bufS = (bufs0, bufs1)
    bufR0 = (bufr00, bufr01)
    bufR1 = (bufr10, bufr11)
    semS = (sems0, sems1)
    semR0 = (semr00, semr01)
    semR1 = (semr10, semr11)

    def start(c, b):
        pa = pos_v[pl.ds(c * 2 * NLANE, NLANE)]
        pb = pos_v[pl.ds(c * 2 * NLANE + NLANE, NLANE)]
        return (
            pltpu.async_copy(ys_hbm.at[pl.ds(tbase + c * NLANE, NLANE)],
                             bufS[b], semS[b]),
            pltpu.async_copy(yr_hbm.at[pa], bufR0[b], semR0[b]),
            pltpu.async_copy(yr_hbm.at[pb], bufR1[b], semR1[b]),
        )

    cops = [None, None]
    cops[0] = start(0, 0)
    for c in range(_CCH):
        b = c % 2
        for hnd in cops[b]:
            hnd.wait()
        if c + 1 < _CCH:
            cops[(c + 1) % 2] = start(c + 1, (c + 1) % 2)
        bs, br0, br1 = bufS[b], bufR0[b], bufR1[b]

        def vsum(j, _):
            for t in range(NLANE):
                br = br0 if t < NLANE // 2 else br1
                u = 2 * t if t < NLANE // 2 else 2 * (t - NLANE // 2)
                bufo[t, pl.ds(j * NLANE, NLANE)] = (
                    bs[t, pl.ds(j * NLANE, NLANE)]
                    + br[u, pl.ds(j * NLANE, NLANE)]
                    + br[u + 1, pl.ds(j * NLANE, NLANE)])
            return 0
        lax.fori_loop(0, _VH, vsum, 0)
        pltpu.sync_copy(bufo, out_hbm.at[pl.ds(tbase + c * NLANE, NLANE)])


def _combine(y_s, y_r, pos):
    return pl.kernel(
        _combine_body,
        out_type=jax.ShapeDtypeStruct((T, H), jnp.float32),
        mesh=_mesh(),
        scratch_types=[
            pltpu.VMEM((_CROWS * 2,), jnp.int32),
            pltpu.VMEM((NLANE, H), jnp.float32),
            pltpu.VMEM((NLANE, H), jnp.float32),
            pltpu.VMEM((NLANE, H), jnp.float32),
            pltpu.VMEM((NLANE, H), jnp.float32),
            pltpu.VMEM((NLANE, H), jnp.float32),
            pltpu.VMEM((NLANE, H), jnp.float32),
            pltpu.VMEM((NLANE, H), jnp.float32),
            pltpu.SemaphoreType.DMA,
            pltpu.SemaphoreType.DMA,
            pltpu.SemaphoreType.DMA,
            pltpu.SemaphoreType.DMA,
            pltpu.SemaphoreType.DMA,
            pltpu.SemaphoreType.DMA,
        ],
        compiler_params=pltpu.CompilerParams(needs_layout_passes=False),
    )(y_s, y_r, pos)


def kernel(x, Wg_s, Wu_s, Wd_s, gate_W, Wg, Wu, Wd):
    x_flat = x.reshape(T, H)
    idx, w = _router(x_flat, gate_W)
    meta, stok, pos, sw = _plan(idx.reshape(-1), w.reshape(-1))
    gx = _gather_x(x_flat, stok)
    y_s = _shared_ff(x_flat, Wg_s, Wu_s, Wd_s)
    y_r = _grouped_ff(meta, gx, sw.reshape(NBLK, 1, BT), Wg, Wu, Wd)
    out = _combine(y_s, y_r, pos)
    return out.reshape(B, S, H)
